# baseline (speedup 1.0000x reference)
import functools

import jax
import jax.numpy as jnp
from jax import lax
from jax.experimental import pallas as pl
from jax.experimental.pallas import tpu as pltpu
from jax.experimental.pallas import tpu_sc as plsc

_LANES = 16
_NBUF = 2
_NCHUNK = 4


@functools.lru_cache(maxsize=None)
def _make_sc_kernel(batch: int, hist: int):
    info = plsc.get_sparse_core_info()
    nw = info.num_cores * info.num_subcores
    assert batch % (nw * _LANES * _NCHUNK) == 0
    rows_w = batch // nw
    rows_c = rows_w // _NCHUNK
    words_c = rows_c * hist
    groups_c = rows_c // _LANES
    mesh = plsc.VectorSubcoreMesh(core_axis_name="c", subcore_axis_name="s")

    @functools.partial(
        pl.kernel,
        out_type=jax.ShapeDtypeStruct((batch,), jnp.float32),
        mesh=mesh,
        scratch_types=[
            pltpu.VMEM((words_c,), jnp.int32),
            pltpu.VMEM((words_c,), jnp.int32),
            pltpu.VMEM((rows_w,), jnp.float32),
            pltpu.VMEM((3, 4), jnp.float32),
            pltpu.VMEM((4, 1), jnp.float32),
            pltpu.VMEM((1,), jnp.float32),
            pltpu.SemaphoreType.DMA((_NBUF,)),
        ],
        compiler_params=pltpu.CompilerParams(needs_layout_passes=False),
    )
    def kern(ids_hbm, t_hbm, w_hbm, b_hbm, out_hbm,
             ids_v0, ids_v1, out_v, t_v, w_v, b_v, sems):
        wid = lax.axis_index("s") * info.num_cores + lax.axis_index("c")
        base = wid * rows_w * hist
        bufs = (ids_v0, ids_v1)
        cps = [
            pltpu.make_async_copy(
                ids_hbm.at[pl.ds(base + c * words_c, words_c)],
                bufs[c % _NBUF], sems.at[c % _NBUF])
            for c in range(_NCHUNK)
        ]
        cps[0].start()
        cps[1].start()
        pltpu.sync_copy(t_hbm, t_v)
        pltpu.sync_copy(w_hbm, w_v)
        pltpu.sync_copy(b_hbm, b_v)

        lane = lax.iota(jnp.int32, _LANES)
        tmask = lane < 12
        wmask = lane < 4
        bmask = lane < 1
        zero = jnp.zeros((_LANES,), jnp.int32)
        tvals = plsc.load_gather(
            t_v, [jnp.where(tmask, lane // 4, 0), lane % 4], mask=tmask)
        wvals = plsc.load_gather(
            w_v, [jnp.where(wmask, lane, 0), zero], mask=wmask)
        bvals = plsc.load_gather(b_v, [zero], mask=bmask)

        def s_of(v):
            acc = tvals[4 * v] * wvals[0]
            for d in range(1, 4):
                acc = acc + tvals[4 * v + d] * wvals[d]
            return acc

        s0, s1, s2 = s_of(0), s_of(1), s_of(2)
        bias = bvals[0]
        beta = s1 - s0
        gamma = 0.5 * (s2 - 2.0 * s1 + s0)
        inv_h = 1.0 / hist
        lane_off = lane * hist

        for c in range(_NCHUNK):
            cps[c].wait()
            if c + _NBUF < _NCHUNK:
                cps[c + _NBUF].start()
            buf = bufs[c % _NBUF]

            def group_body(g, _, buf=buf, c=c):
                idx0 = g * (_LANES * hist) + lane_off
                acc1 = jnp.zeros((_LANES,), jnp.int32)
                acc2 = jnp.zeros((_LANES,), jnp.int32)
                for j in range(hist):
                    v = plsc.load_gather(buf, [idx0 + j])
                    acc1 = acc1 + v
                    acc2 = acc2 + v * v
                f1 = acc1.astype(jnp.float32)
                f2 = acc2.astype(jnp.float32)
                logit = s0 + (beta * f1 + gamma * (f2 - f1)) * inv_h + bias
                out_v[pl.ds((c * groups_c + g) * _LANES, _LANES)] = (
                    1.0 / (1.0 + jnp.exp(-logit)))
                return _

            lax.fori_loop(0, groups_c, group_body, None)

        pltpu.sync_copy(out_v, out_hbm.at[pl.ds(wid * rows_w, rows_w)])

    return kern


def kernel(color_ids, table, W, b):
    batch, hist = color_ids.shape
    ids_flat = color_ids.astype(jnp.int32).reshape(-1)
    out = _make_sc_kernel(batch, hist)(
        ids_flat, table.astype(jnp.float32), W.astype(jnp.float32),
        b.astype(jnp.float32))
    return out.reshape(batch, 1)


# speedup vs baseline: 1.0307x; 1.0307x over previous
import functools

import jax
import jax.numpy as jnp
from jax import lax
from jax.experimental import pallas as pl
from jax.experimental.pallas import tpu as pltpu
from jax.experimental.pallas import tpu_sc as plsc

_LANES = 16


@functools.lru_cache(maxsize=None)
def _make_sc_kernel(batch: int, hist: int):
    info = plsc.get_sparse_core_info()
    nw = info.num_cores * info.num_subcores
    assert batch % (nw * _LANES) == 0
    rows_w = batch // nw
    words_w = rows_w * hist
    groups = rows_w // _LANES
    mesh = plsc.VectorSubcoreMesh(core_axis_name="c", subcore_axis_name="s")

    @functools.partial(
        pl.kernel,
        out_type=jax.ShapeDtypeStruct((batch,), jnp.float32),
        mesh=mesh,
        scratch_types=[
            pltpu.VMEM((words_w,), jnp.int32),
            pltpu.VMEM((rows_w,), jnp.float32),
            pltpu.VMEM((32,), jnp.float32),
        ],
        compiler_params=pltpu.CompilerParams(
            needs_layout_passes=False,
            skip_device_barrier=True,
            disable_bounds_checks=True,
            disable_semaphore_checks=True,
        ),
    )
    def kern(ids_hbm, par_hbm, out_hbm, ids_v, out_v, par_v):
        wid = lax.axis_index("s") * info.num_cores + lax.axis_index("c")
        base = wid * words_w
        pltpu.sync_copy(par_hbm, par_v)
        pltpu.sync_copy(ids_hbm.at[pl.ds(base, words_w)], ids_v)

        p0 = par_v[pl.ds(0, _LANES)]
        p1 = par_v[pl.ds(_LANES, _LANES)]

        def s_of(v):
            acc = p0[4 * v] * p0[12]
            for d in range(1, 4):
                acc = acc + p0[4 * v + d] * p0[12 + d]
            return acc

        s0, s1, s2 = s_of(0), s_of(1), s_of(2)
        bias = p1[0]
        beta = s1 - s0
        gamma = 0.5 * (s2 - 2.0 * s1 + s0)
        inv_h = 1.0 / hist
        lane_off = lax.iota(jnp.int32, _LANES) * hist

        def group_body(g, _):
            idx0 = g * (_LANES * hist) + lane_off
            acc1 = jnp.zeros((_LANES,), jnp.int32)
            acc2 = jnp.zeros((_LANES,), jnp.int32)
            for j in range(hist):
                v = plsc.load_gather(ids_v, [idx0 + j])
                acc1 = acc1 + v
                acc2 = acc2 + v * v
            f1 = acc1.astype(jnp.float32)
            f2 = acc2.astype(jnp.float32)
            logit = s0 + (beta * f1 + gamma * (f2 - f1)) * inv_h + bias
            out_v[pl.ds(g * _LANES, _LANES)] = 1.0 / (1.0 + jnp.exp(-logit))
            return _

        lax.fori_loop(0, groups, group_body, None)
        pltpu.sync_copy(out_v, out_hbm.at[pl.ds(wid * rows_w, rows_w)])

    return kern


def kernel(color_ids, table, W, b):
    batch, hist = color_ids.shape
    params = jnp.concatenate([
        table.reshape(-1).astype(jnp.float32),
        W.reshape(-1).astype(jnp.float32),
        b.reshape(-1).astype(jnp.float32),
        jnp.zeros((15,), jnp.float32),
    ])
    ids_flat = color_ids.astype(jnp.int32).reshape(-1)
    out = _make_sc_kernel(batch, hist)(ids_flat, params)
    return out.reshape(batch, 1)


# speedup vs baseline: 1.0364x; 1.0055x over previous
import functools

import jax
import jax.numpy as jnp
from jax import lax
from jax.experimental import pallas as pl
from jax.experimental.pallas import tpu as pltpu
from jax.experimental.pallas import tpu_sc as plsc

_LANES = 16


@functools.lru_cache(maxsize=None)
def _make_sc_kernel(batch: int, hist: int):
    info = plsc.get_sparse_core_info()
    nw = info.num_cores * info.num_subcores
    assert batch % (nw * _LANES) == 0
    rows_w = batch // nw
    groups = rows_w // _LANES
    mesh = plsc.VectorSubcoreMesh(core_axis_name="c", subcore_axis_name="s")

    @functools.partial(
        pl.kernel,
        out_type=jax.ShapeDtypeStruct((batch,), jnp.float32),
        mesh=mesh,
        scratch_types=[
            pltpu.VMEM((rows_w, hist), jnp.int32),
            pltpu.VMEM((rows_w,), jnp.float32),
            pltpu.VMEM((32,), jnp.float32),
        ],
        compiler_params=pltpu.CompilerParams(
            needs_layout_passes=False,
            use_tc_tiling_on_sc=True,
        ),
    )
    def kern(ids_hbm, par_hbm, out_hbm, ids_v, out_v, par_v):
        wid = lax.axis_index("s") * info.num_cores + lax.axis_index("c")
        rowbase = wid * rows_w
        pltpu.sync_copy(par_hbm, par_v)
        pltpu.sync_copy(ids_hbm.at[pl.ds(rowbase, rows_w), :], ids_v)

        p0 = par_v[pl.ds(0, _LANES)]
        p1 = par_v[pl.ds(_LANES, _LANES)]

        def s_of(v):
            acc = p0[4 * v] * p0[12]
            for d in range(1, 4):
                acc = acc + p0[4 * v + d] * p0[12 + d]
            return acc

        s0, s1, s2 = s_of(0), s_of(1), s_of(2)
        bias = p1[0]
        beta = s1 - s0
        gamma = 0.5 * (s2 - 2.0 * s1 + s0)
        inv_h = 1.0 / hist
        lane = lax.iota(jnp.int32, _LANES)

        def group_body(g, _):
            rows = g * _LANES + lane
            acc1 = jnp.zeros((_LANES,), jnp.int32)
            acc2 = jnp.zeros((_LANES,), jnp.int32)
            for j in range(hist):
                v = plsc.load_gather(ids_v, [rows, jnp.full((_LANES,), j, jnp.int32)])
                acc1 = acc1 + v
                acc2 = acc2 + v * v
            f1 = acc1.astype(jnp.float32)
            f2 = acc2.astype(jnp.float32)
            logit = s0 + (beta * f1 + gamma * (f2 - f1)) * inv_h + bias
            out_v[pl.ds(g * _LANES, _LANES)] = 1.0 / (1.0 + jnp.exp(-logit))
            return _

        lax.fori_loop(0, groups, group_body, None)
        pltpu.sync_copy(out_v, out_hbm.at[pl.ds(rowbase, rows_w)])

    return kern


def kernel(color_ids, table, W, b):
    batch, hist = color_ids.shape
    params = jnp.concatenate([
        table.reshape(-1).astype(jnp.float32),
        W.reshape(-1).astype(jnp.float32),
        b.reshape(-1).astype(jnp.float32),
        jnp.zeros((15,), jnp.float32),
    ])
    out = _make_sc_kernel(batch, hist)(color_ids.astype(jnp.int32), params)
    return out.reshape(batch, 1)


# speedup vs baseline: 1.7534x; 1.6919x over previous
import functools

import jax
import jax.numpy as jnp
from jax import lax
from jax.experimental import pallas as pl
from jax.experimental.pallas import tpu as pltpu
from jax.experimental.pallas import tpu_sc as plsc

_LANES = 16


@functools.lru_cache(maxsize=None)
def _make_sc_kernel(batch: int, hist: int):
    info = plsc.get_sparse_core_info()
    nw = info.num_cores * info.num_subcores
    assert batch % (nw * _LANES) == 0
    rows_w = batch // nw
    groups = rows_w // _LANES
    mesh = plsc.VectorSubcoreMesh(core_axis_name="c", subcore_axis_name="s")

    @functools.partial(
        pl.kernel,
        out_type=jax.ShapeDtypeStruct((batch,), jnp.float32),
        mesh=mesh,
        scratch_types=[
            pltpu.VMEM((hist, rows_w), jnp.int32),
            pltpu.VMEM((rows_w,), jnp.float32),
            pltpu.VMEM((32,), jnp.float32),
        ],
        compiler_params=pltpu.CompilerParams(
            needs_layout_passes=False,
            use_tc_tiling_on_sc=True,
        ),
    )
    def kern(idsT_hbm, par_hbm, out_hbm, ids_v, out_v, par_v):
        wid = lax.axis_index("s") * info.num_cores + lax.axis_index("c")
        base = wid * rows_w
        pltpu.sync_copy(par_hbm, par_v)
        pltpu.sync_copy(idsT_hbm.at[:, pl.ds(base, rows_w)], ids_v)

        p0 = par_v[pl.ds(0, _LANES)]
        p1 = par_v[pl.ds(_LANES, _LANES)]

        def s_of(v):
            acc = p0[4 * v] * p0[12]
            for d in range(1, 4):
                acc = acc + p0[4 * v + d] * p0[12 + d]
            return acc

        s0, s1, s2 = s_of(0), s_of(1), s_of(2)
        bias = p1[0]
        beta = s1 - s0
        gamma = 0.5 * (s2 - 2.0 * s1 + s0)
        inv_h = 1.0 / hist

        def group_body(g, _):
            col = g * _LANES
            acc1 = jnp.zeros((_LANES,), jnp.int32)
            acc2 = jnp.zeros((_LANES,), jnp.int32)
            for j in range(hist):
                v = ids_v[j, pl.ds(col, _LANES)]
                acc1 = acc1 + v
                acc2 = acc2 + v * v
            f1 = acc1.astype(jnp.float32)
            f2 = acc2.astype(jnp.float32)
            logit = s0 + (beta * f1 + gamma * (f2 - f1)) * inv_h + bias
            out_v[pl.ds(col, _LANES)] = 1.0 / (1.0 + jnp.exp(-logit))
            return _

        lax.fori_loop(0, groups, group_body, None)
        pltpu.sync_copy(out_v, out_hbm.at[pl.ds(base, rows_w)])

    return kern


def kernel(color_ids, table, W, b):
    batch, hist = color_ids.shape
    params = jnp.concatenate([
        table.reshape(-1).astype(jnp.float32),
        W.reshape(-1).astype(jnp.float32),
        b.reshape(-1).astype(jnp.float32),
        jnp.zeros((15,), jnp.float32),
    ])
    ids_t = color_ids.astype(jnp.int32).T
    out = _make_sc_kernel(batch, hist)(ids_t, params)
    return out.reshape(batch, 1)
